# baseline (speedup 1.0000x reference)
import functools

import jax
import jax.numpy as jnp
from jax import lax
from jax.experimental import pallas as pl
from jax.experimental.pallas import tpu as pltpu
from jax.experimental.pallas import tpu_sc as plsc

C = 1000
S = 1000
B = 16384
ROWS_PER_BLOCK = 1024
NUM_BLOCKS = B // ROWS_PER_BLOCK

NC = 2
NS = 16
NW = NC * NS
BPW = B // NW

ENTRIES_PER_ROW = 64
ROW_WORDS = 2 * ENTRIES_PER_ROW
TROWS = C * S // ENTRIES_PER_ROW


CHUNK = 256
NCHUNKS = B // CHUNK
NBUF = 8


def _flat_idx_body(attr_hbm, coarse_ref, col_ref, buf, sems):
    def copy_op(g, phase):
        return pltpu.make_async_copy(
            attr_hbm.at[pl.ds(g * CHUNK, CHUNK), :],
            buf.at[phase],
            sems.at[phase],
        )

    for g in range(NBUF):
        copy_op(g, g).start()

    col = lax.broadcasted_iota(jnp.int32, (1, C + S), 1)
    w = jnp.where(col < C, col * S, col - C).astype(jnp.float32)

    def outer(o, _):
        for phase in range(NBUF):
            g = o * NBUF + phase
            copy_op(g, phase).wait()
            a = buf[phase]
            flat = jnp.sum(a * w, axis=1).astype(jnp.int32)
            sl = pl.ds(g * CHUNK, CHUNK)
            coarse_ref[sl] = lax.shift_right_logical(flat, 6)
            col_ref[sl] = lax.shift_left(jnp.bitwise_and(flat, 63), 1)

            @pl.when(g + NBUF < NCHUNKS)
            def _():
                copy_op(g + NBUF, phase).start()
        return None

    lax.fori_loop(0, NCHUNKS // NBUF, outer, None)


def _flat_idx_tc(attr):
    return pl.pallas_call(
        _flat_idx_body,
        in_specs=[pl.BlockSpec(memory_space=pl.ANY)],
        out_specs=[
            pl.BlockSpec(memory_space=pltpu.VMEM),
            pl.BlockSpec(memory_space=pltpu.VMEM),
        ],
        out_shape=[
            jax.ShapeDtypeStruct((B,), jnp.int32),
            jax.ShapeDtypeStruct((B,), jnp.int32),
        ],
        scratch_shapes=[
            pltpu.VMEM((NBUF, CHUNK, C + S), jnp.float32),
            pltpu.SemaphoreType.DMA((NBUF,)),
        ],
        compiler_params=pltpu.CompilerParams(
            vmem_limit_bytes=100 * 1024 * 1024,
        ),
    )(attr)


@functools.cache
def _make_gather_sc():
    mesh = plsc.VectorSubcoreMesh(core_axis_name="c", subcore_axis_name="s")
    return pl.kernel(
        _gather_sc_body,
        mesh=mesh,
        out_type=jax.ShapeDtypeStruct((B, ROW_WORDS), jnp.int32),
        name="vocab_gather_sc",
        scratch_types=[
            pltpu.VMEM((BPW,), jnp.int32),
            pltpu.VMEM((BPW, ROW_WORDS), jnp.int32),
            pltpu.SemaphoreType.DMA,
        ],
    )


def _gather_sc_body(table_hbm, coarse_hbm, out_hbm, coarse_v, rows_v, sem):
    wid = lax.axis_index("s") * NC + lax.axis_index("c")
    base = wid * BPW
    pltpu.sync_copy(coarse_hbm.at[wid], coarse_v)
    pltpu.async_copy(table_hbm.at[coarse_v], rows_v, sem).wait()
    pltpu.sync_copy(rows_v, out_hbm.at[pl.ds(base, BPW)])


TBLOCK = TROWS


def _identity_body(in_ref, out_ref):
    out_ref[...] = in_ref[...]


def _table_to_tc(table):
    return pl.pallas_call(
        _identity_body,
        grid=(TROWS // TBLOCK,),
        in_specs=[pl.BlockSpec((TBLOCK, ROW_WORDS), lambda i: (i, 0))],
        out_specs=pl.BlockSpec((TBLOCK, ROW_WORDS), lambda i: (i, 0)),
        out_shape=jax.ShapeDtypeStruct((TROWS, ROW_WORDS), jnp.int32),
        compiler_params=pltpu.CompilerParams(
            dimension_semantics=("arbitrary",),
            vmem_limit_bytes=100 * 1024 * 1024,
        ),
    )(table)


def _extract_body(rows_ref, col_ref, out_ref):
    a = rows_ref[...]
    c0 = col_ref[...][:, None]
    lane = lax.broadcasted_iota(jnp.int32, (1, ROW_WORDS), 1)
    v0 = jnp.sum(jnp.where(lane == c0, a, 0), axis=1)
    v1 = jnp.sum(jnp.where(lane == c0 + 1, a, 0), axis=1)
    out_ref[...] = jnp.concatenate([v0[:, None], v1[:, None]], axis=1)


EBLOCK = 512


def _extract_tc(rows, cols):
    return pl.pallas_call(
        _extract_body,
        grid=(B // EBLOCK,),
        in_specs=[
            pl.BlockSpec((EBLOCK, ROW_WORDS), lambda i: (i, 0)),
            pl.BlockSpec((EBLOCK,), lambda i: (i,)),
        ],
        out_specs=pl.BlockSpec((EBLOCK, 2), lambda i: (i, 0)),
        out_shape=jax.ShapeDtypeStruct((B, 2), jnp.int32),
        compiler_params=pltpu.CompilerParams(
            dimension_semantics=("arbitrary",),
        ),
    )(rows, cols)


def kernel(attrVector, permVocab):
    coarse, cols = _flat_idx_tc(attrVector)
    table = _table_to_tc(permVocab.reshape(TROWS, ROW_WORDS))
    rows = _make_gather_sc()(table, coarse.reshape(NW, BPW))
    perm_messages = _extract_tc(rows, cols)
    z = jnp.zeros((B,), dtype=jnp.float32)
    return (perm_messages, z, z, jnp.ones((B,), dtype=jnp.float32))


# speedup vs baseline: 1.0019x; 1.0019x over previous
import functools

import jax
import jax.numpy as jnp
from jax import lax
from jax.experimental import pallas as pl
from jax.experimental.pallas import tpu as pltpu
from jax.experimental.pallas import tpu_sc as plsc

C = 1000
S = 1000
B = 16384
ROWS_PER_BLOCK = 1024
NUM_BLOCKS = B // ROWS_PER_BLOCK

NC = 2
NS = 16
NW = NC * NS
BPW = B // NW

ENTRIES_PER_ROW = 64
ROW_WORDS = 2 * ENTRIES_PER_ROW
TROWS = C * S // ENTRIES_PER_ROW


CHUNK = 256
NCHUNKS = B // CHUNK
NBUF = 8


def _flat_idx_body(attr_hbm, coarse_ref, col_ref, buf, sems):
    def copy_op(g, phase):
        return pltpu.make_async_copy(
            attr_hbm.at[pl.ds(g * CHUNK, CHUNK), :],
            buf.at[phase],
            sems.at[phase],
        )

    for g in range(NBUF):
        copy_op(g, g).start()

    col = lax.broadcasted_iota(jnp.int32, (1, C + S), 1)
    w = jnp.where(col < C, col * S, col - C).astype(jnp.float32)

    def outer(o, _):
        for phase in range(NBUF):
            g = o * NBUF + phase
            copy_op(g, phase).wait()
            a = buf[phase]
            flat = jnp.sum(a * w, axis=1).astype(jnp.int32)
            sl = pl.ds(g * CHUNK, CHUNK)
            coarse_ref[sl] = lax.shift_right_logical(flat, 6)
            col_ref[sl] = lax.shift_left(jnp.bitwise_and(flat, 63), 1)

            @pl.when(g + NBUF < NCHUNKS)
            def _():
                copy_op(g + NBUF, phase).start()
        return None

    lax.fori_loop(0, NCHUNKS // NBUF, outer, None)


def _flat_idx_tc(attr):
    return pl.pallas_call(
        _flat_idx_body,
        in_specs=[pl.BlockSpec(memory_space=pl.ANY)],
        out_specs=[
            pl.BlockSpec(memory_space=pltpu.VMEM),
            pl.BlockSpec(memory_space=pltpu.VMEM),
        ],
        out_shape=[
            jax.ShapeDtypeStruct((B,), jnp.int32),
            jax.ShapeDtypeStruct((B,), jnp.int32),
        ],
        scratch_shapes=[
            pltpu.VMEM((NBUF, CHUNK, C + S), jnp.float32),
            pltpu.SemaphoreType.DMA((NBUF,)),
        ],
        compiler_params=pltpu.CompilerParams(
            vmem_limit_bytes=100 * 1024 * 1024,
        ),
    )(attr)


@functools.cache
def _make_gather_sc():
    mesh = plsc.VectorSubcoreMesh(core_axis_name="c", subcore_axis_name="s")
    return pl.kernel(
        _gather_sc_body,
        mesh=mesh,
        out_type=jax.ShapeDtypeStruct((B, ROW_WORDS), jnp.int32),
        name="vocab_gather_sc",
        scratch_types=[
            pltpu.VMEM((BPW,), jnp.int32),
            pltpu.VMEM((BPW, ROW_WORDS), jnp.int32),
            pltpu.SemaphoreType.DMA,
        ],
    )


def _gather_sc_body(table_hbm, coarse_hbm, out_hbm, coarse_v, rows_v, sem):
    wid = lax.axis_index("s") * NC + lax.axis_index("c")
    base = wid * BPW
    pltpu.sync_copy(coarse_hbm.at[wid], coarse_v)
    pltpu.async_copy(table_hbm.at[coarse_v], rows_v, sem).wait()
    pltpu.sync_copy(rows_v, out_hbm.at[pl.ds(base, BPW)])


def _extract_body(rows_ref, col_ref, out_ref):
    a = rows_ref[...]
    c0 = col_ref[...][:, None]
    lane = lax.broadcasted_iota(jnp.int32, (1, ROW_WORDS), 1)
    v0 = jnp.sum(jnp.where(lane == c0, a, 0), axis=1)
    v1 = jnp.sum(jnp.where(lane == c0 + 1, a, 0), axis=1)
    out_ref[...] = jnp.concatenate([v0[:, None], v1[:, None]], axis=1)


EBLOCK = 512


def _extract_tc(rows, cols):
    return pl.pallas_call(
        _extract_body,
        grid=(B // EBLOCK,),
        in_specs=[
            pl.BlockSpec((EBLOCK, ROW_WORDS), lambda i: (i, 0)),
            pl.BlockSpec((EBLOCK,), lambda i: (i,)),
        ],
        out_specs=pl.BlockSpec((EBLOCK, 2), lambda i: (i, 0)),
        out_shape=jax.ShapeDtypeStruct((B, 2), jnp.int32),
        compiler_params=pltpu.CompilerParams(
            dimension_semantics=("arbitrary",),
        ),
    )(rows, cols)


def kernel(attrVector, permVocab):
    coarse, cols = _flat_idx_tc(attrVector)
    table = permVocab.reshape(TROWS, ROW_WORDS)
    rows = _make_gather_sc()(table, coarse.reshape(NW, BPW))
    perm_messages = _extract_tc(rows, cols)
    z = jnp.zeros((B,), dtype=jnp.float32)
    return (perm_messages, z, z, jnp.ones((B,), dtype=jnp.float32))
